# baseline (speedup 1.0000x reference)
import jax
import jax.numpy as jnp
from jax.experimental import pallas as pl
from jax.experimental.pallas import tpu as pltpu

_NUM_RING = 4
_EPS = 1e-5
_DO = 128
_DR = _NUM_RING * _DO
_NEG = -1e30


def _stats_kernel(x_ref, r_ref, w_ref, m_ref, s1_ref, s2_ref, cnt_ref):
    b = pl.program_id(0)
    nt = pl.program_id(1)

    @pl.when(nt == 0)
    def _init_max():
        m_ref[0] = jnp.full(m_ref.shape[1:], _NEG, jnp.float32)

    @pl.when(jnp.logical_and(b == 0, nt == 0))
    def _init_sums():
        s1_ref[...] = jnp.zeros(s1_ref.shape, jnp.float32)
        s2_ref[...] = jnp.zeros(s2_ref.shape, jnp.float32)
        cnt_ref[...] = jnp.zeros(cnt_ref.shape, jnp.float32)

    xb = x_ref[0]
    z = jax.lax.dot_general(
        w_ref[...], xb, (((1,), (0,)), ((), ())),
        preferred_element_type=jnp.float32)
    zz = z * z
    r = r_ref[0]

    ohf = jnp.concatenate(
        [jnp.where(r == i, 1.0, 0.0) for i in range(_NUM_RING)], axis=0)

    dn = (((1,), (1,)), ((), ()))
    s1c = jax.lax.dot_general(z, ohf, dn, preferred_element_type=jnp.float32)
    s2c = jax.lax.dot_general(zz, ohf, dn, preferred_element_type=jnp.float32)
    cntc = jax.lax.dot_general(ohf, ohf, dn, preferred_element_type=jnp.float32)
    s1_ref[:, 0:_NUM_RING] += s1c
    s2_ref[:, 0:_NUM_RING] += s2c
    cnt_ref[0:_NUM_RING, 0:_NUM_RING] += cntc

    pen = jnp.concatenate(
        [jnp.broadcast_to(jnp.where(r == i, 0.0, _NEG), (_DO, r.shape[1]))
         for i in range(_NUM_RING)], axis=0)
    pmax = jnp.max(z + pen, axis=1, keepdims=True)
    m_ref[0] = jnp.maximum(m_ref[0], pmax)


def _bcast_kernel(m_ref, s1_ref, s2_ref, cnt_ref, bb_ref, gb_ref, be_ref,
                  r_ref, out_ref, mx_ref):
    ni = pl.program_id(1)

    @pl.when(ni == 0)
    def _affine():
        rowblk = jax.lax.broadcasted_iota(jnp.int32, (_DR, 1), 0) // _DO
        zero = jnp.zeros((_DR, 1), jnp.float32)
        s1 = zero
        s2 = zero
        cnt = zero
        for i in range(_NUM_RING):
            blk = rowblk == i
            s1 = jnp.where(blk, s1_ref[:, i:i + 1], s1)
            s2 = jnp.where(blk, s2_ref[:, i:i + 1], s2)
            cnt = jnp.where(blk, cnt_ref[i, i], cnt)
        bb = bb_ref[:, 0:1]
        gb = gb_ref[:, 0:1]
        be = be_ref[:, 0:1]
        cmax = jnp.maximum(cnt, 1.0)
        s1y = s1 + cnt * bb
        s2y = s2 + 2.0 * bb * s1 + cnt * bb * bb
        mean = s1y / cmax
        var = s2y / cmax - mean * mean
        inv = jax.lax.rsqrt(var + _EPS)
        mx = (m_ref[0][:, 0:1] + bb - mean) * (inv * gb) + be
        for i in range(_NUM_RING):
            mx_ref[:, i:i + 1] = mx[i * _DO:(i + 1) * _DO, :]

    r = r_ref[0]
    ohf = jnp.concatenate(
        [jnp.where(r == i, 1.0, 0.0) for i in range(_NUM_RING)], axis=0)
    out_ref[0] = jax.lax.dot_general(
        mx_ref[:, 0:_NUM_RING], ohf, (((1,), (0,)), ((), ())),
        preferred_element_type=jnp.float32)


def kernel(x, ring, W, b, gamma, beta):
    B_, D, N = x.shape
    ring3 = ring.reshape(B_, 1, N)
    wcat = W.reshape(_DR, D)
    bb = jnp.broadcast_to(b.reshape(_DR, 1), (_DR, 8))
    gb = jnp.broadcast_to(gamma.reshape(_DR, 1), (_DR, 8))
    be = jnp.broadcast_to(beta.reshape(_DR, 1), (_DR, 8))

    TN = 1024
    nt = N // TN
    small = jax.ShapeDtypeStruct((_DR, 8), jnp.float32)
    cst = lambda shape: pl.BlockSpec(shape, lambda bi, ni: tuple(0 for _ in shape))
    M, S1, S2, CNT = pl.pallas_call(
        _stats_kernel,
        grid=(B_, nt),
        in_specs=[
            pl.BlockSpec((1, D, TN), lambda bi, ni: (bi, 0, ni)),
            pl.BlockSpec((1, 1, TN), lambda bi, ni: (bi, 0, ni)),
            cst((_DR, D)),
        ],
        out_specs=[
            pl.BlockSpec((1, _DR, 8), lambda bi, ni: (bi, 0, 0)),
            cst((_DR, 8)),
            cst((_DR, 8)),
            cst((8, 8)),
        ],
        out_shape=[
            jax.ShapeDtypeStruct((B_, _DR, 8), jnp.float32),
            small, small,
            jax.ShapeDtypeStruct((8, 8), jnp.float32),
        ],
        compiler_params=pltpu.CompilerParams(
            dimension_semantics=("arbitrary", "arbitrary")),
    )(x, ring3, wcat)

    TN2 = 2048
    nt2 = N // TN2
    out = pl.pallas_call(
        _bcast_kernel,
        grid=(B_, nt2),
        in_specs=[
            pl.BlockSpec((1, _DR, 8), lambda bi, ni: (bi, 0, 0)),
            cst((_DR, 8)),
            cst((_DR, 8)),
            cst((8, 8)),
            cst((_DR, 8)),
            cst((_DR, 8)),
            cst((_DR, 8)),
            pl.BlockSpec((1, 1, TN2), lambda bi, ni: (bi, 0, ni)),
        ],
        out_specs=pl.BlockSpec((1, _DO, TN2), lambda bi, ni: (bi, 0, ni)),
        out_shape=jax.ShapeDtypeStruct((B_, _DO, N), jnp.float32),
        scratch_shapes=[pltpu.VMEM((_DO, 8), jnp.float32)],
        compiler_params=pltpu.CompilerParams(
            dimension_semantics=("arbitrary", "arbitrary")),
    )(M, S1, S2, CNT, bb, gb, be, ring3)
    return out


# speedup vs baseline: 1.2002x; 1.2002x over previous
import jax
import jax.numpy as jnp
from jax.experimental import pallas as pl
from jax.experimental.pallas import tpu as pltpu

_NUM_RING = 4
_EPS = 1e-5
_DO = 128
_DR = _NUM_RING * _DO
_NEG = -1e30


def _stats_kernel(x_ref, r_ref, w_ref, m_ref, s1_ref, s2_ref, cnt_ref):
    b = pl.program_id(0)
    nt = pl.program_id(1)

    @pl.when(nt == 0)
    def _init_max():
        m_ref[0] = jnp.full(m_ref.shape[1:], _NEG, jnp.float32)

    @pl.when(jnp.logical_and(b == 0, nt == 0))
    def _init_sums():
        s1_ref[...] = jnp.zeros(s1_ref.shape, jnp.float32)
        s2_ref[...] = jnp.zeros(s2_ref.shape, jnp.float32)
        cnt_ref[...] = jnp.zeros(cnt_ref.shape, jnp.float32)

    xb = x_ref[0]
    z = jax.lax.dot_general(
        w_ref[...], xb, (((1,), (0,)), ((), ())),
        preferred_element_type=jnp.float32)
    r = r_ref[0]
    for i in range(_NUM_RING):
        mask = r == i
        sl = slice(i * _DO, (i + 1) * _DO)
        zi = z[sl, :]
        u = jnp.where(mask, zi, 0.0)
        pmax = jnp.max(jnp.where(mask, zi, _NEG), axis=1, keepdims=True)
        ps1 = jnp.sum(u, axis=1, keepdims=True)
        ps2 = jnp.sum(u * u, axis=1, keepdims=True)
        pc = jnp.sum(jnp.where(mask, 1.0, 0.0), axis=1, keepdims=True)
        m_ref[0, sl, :] = jnp.maximum(m_ref[0, sl, :], pmax)
        s1_ref[sl, :] = s1_ref[sl, :] + ps1
        s2_ref[sl, :] = s2_ref[sl, :] + ps2
        cnt_ref[i:i + 1, 0:1] = cnt_ref[i:i + 1, 0:1] + pc


def _bcast_kernel(m_ref, s1_ref, s2_ref, cnt_ref, bb_ref, gb_ref, be_ref,
                  r_ref, out_ref, mx_ref):
    ni = pl.program_id(1)

    @pl.when(ni == 0)
    def _affine():
        cnt = jnp.concatenate(
            [jnp.broadcast_to(cnt_ref[i:i + 1, 0:1], (_DO, 1))
             for i in range(_NUM_RING)], axis=0)
        s1 = s1_ref[:, 0:1]
        s2 = s2_ref[:, 0:1]
        bb = bb_ref[:, 0:1]
        gb = gb_ref[:, 0:1]
        be = be_ref[:, 0:1]
        cmax = jnp.maximum(cnt, 1.0)
        s1y = s1 + cnt * bb
        s2y = s2 + 2.0 * bb * s1 + cnt * bb * bb
        mean = s1y / cmax
        var = s2y / cmax - mean * mean
        inv = jax.lax.rsqrt(var + _EPS)
        mx_ref[:, 0:1] = (m_ref[0][:, 0:1] + bb - mean) * (inv * gb) + be

    r = r_ref[0]
    acc = jnp.zeros((_DO, r.shape[1]), jnp.float32)
    for i in range(_NUM_RING):
        col = mx_ref[i * _DO:(i + 1) * _DO, 0:1]
        acc = jnp.where(r == i, col, acc)
    out_ref[0] = acc


def kernel(x, ring, W, b, gamma, beta):
    B_, D, N = x.shape
    ring3 = ring.reshape(B_, 1, N)
    wcat = W.reshape(_DR, D)
    bb = jnp.broadcast_to(b.reshape(_DR, 1), (_DR, 8))
    gb = jnp.broadcast_to(gamma.reshape(_DR, 1), (_DR, 8))
    be = jnp.broadcast_to(beta.reshape(_DR, 1), (_DR, 8))

    TN = 1024
    nt = N // TN
    small = jax.ShapeDtypeStruct((_DR, 8), jnp.float32)
    cst = lambda shape: pl.BlockSpec(shape, lambda bi, ni: tuple(0 for _ in shape))
    M, S1, S2, CNT = pl.pallas_call(
        _stats_kernel,
        grid=(B_, nt),
        in_specs=[
            pl.BlockSpec((1, D, TN), lambda bi, ni: (bi, 0, ni)),
            pl.BlockSpec((1, 1, TN), lambda bi, ni: (bi, 0, ni)),
            cst((_DR, D)),
        ],
        out_specs=[
            pl.BlockSpec((1, _DR, 8), lambda bi, ni: (bi, 0, 0)),
            cst((_DR, 8)),
            cst((_DR, 8)),
            cst((8, 8)),
        ],
        out_shape=[
            jax.ShapeDtypeStruct((B_, _DR, 8), jnp.float32),
            small, small,
            jax.ShapeDtypeStruct((8, 8), jnp.float32),
        ],
        compiler_params=pltpu.CompilerParams(
            dimension_semantics=("arbitrary", "arbitrary")),
    )(x, ring3, wcat)

    TN2 = 2048
    nt2 = N // TN2
    out = pl.pallas_call(
        _bcast_kernel,
        grid=(B_, nt2),
        in_specs=[
            pl.BlockSpec((1, _DR, 8), lambda bi, ni: (bi, 0, 0)),
            cst((_DR, 8)),
            cst((_DR, 8)),
            cst((8, 8)),
            cst((_DR, 8)),
            cst((_DR, 8)),
            cst((_DR, 8)),
            pl.BlockSpec((1, 1, TN2), lambda bi, ni: (bi, 0, ni)),
        ],
        out_specs=pl.BlockSpec((1, _DO, TN2), lambda bi, ni: (bi, 0, ni)),
        out_shape=jax.ShapeDtypeStruct((B_, _DO, N), jnp.float32),
        scratch_shapes=[pltpu.VMEM((_DR, 8), jnp.float32)],
        compiler_params=pltpu.CompilerParams(
            dimension_semantics=("arbitrary", "arbitrary")),
    )(M, S1, S2, CNT, bb, gb, be, ring3)
    return out


# speedup vs baseline: 1.2444x; 1.0368x over previous
import jax
import jax.numpy as jnp
from jax.experimental import pallas as pl
from jax.experimental.pallas import tpu as pltpu

_NUM_RING = 4
_EPS = 1e-5
_DO = 128
_DR = _NUM_RING * _DO
_NEG = -1e30


def _stats_kernel(x_ref, r_ref, w_ref, m_ref, sx_ref, c_ref, cnt_ref):
    b = pl.program_id(0)
    nt = pl.program_id(1)

    @pl.when(nt == 0)
    def _init_max():
        m_ref[0] = jnp.full(m_ref.shape[1:], _NEG, jnp.float32)

    @pl.when(jnp.logical_and(b == 0, nt == 0))
    def _init_sums():
        sx_ref[...] = jnp.zeros(sx_ref.shape, jnp.float32)
        c_ref[...] = jnp.zeros(c_ref.shape, jnp.float32)
        cnt_ref[...] = jnp.zeros(cnt_ref.shape, jnp.float32)

    xb = x_ref[0]
    z = jax.lax.dot_general(
        w_ref[...], xb, (((1,), (0,)), ((), ())),
        preferred_element_type=jnp.float32)
    r = r_ref[0]

    for i in range(_NUM_RING):
        sl = slice(i * _DO, (i + 1) * _DO)
        pmax = jnp.max(jnp.where(r == i, z[sl, :], _NEG), axis=1,
                       keepdims=True)
        m_ref[0, sl, :] = jnp.maximum(m_ref[0, sl, :], pmax)

    xbt = jnp.transpose(xb)
    for i in range(_NUM_RING):
        mask = r == i
        xm = jnp.where(mask, xb, 0.0)
        sxi = jnp.sum(xm, axis=1, keepdims=True)
        ci = jax.lax.dot_general(
            xm, xbt, (((1,), (0,)), ((), ())),
            preferred_element_type=jnp.float32)
        pc = jnp.sum(jnp.where(mask, 1.0, 0.0), axis=1, keepdims=True)
        rs = slice(64 * i, 64 * (i + 1))
        sx_ref[rs, 0:1] += sxi
        c_ref[rs, :] += ci
        cnt_ref[i:i + 1, 0:1] += pc


def _bcast_kernel(m_ref, sx_ref, c_ref, cnt_ref, w_ref, bb_ref, gb_ref,
                  be_ref, r_ref, out_ref, mx_ref):
    ni = pl.program_id(1)

    @pl.when(ni == 0)
    def _affine():
        w = w_ref[...]
        s1l, s2l, cntl = [], [], []
        for i in range(_NUM_RING):
            wi = w[i * _DO:(i + 1) * _DO, :]
            rs = slice(64 * i, 64 * (i + 1))
            sxr = jnp.transpose(sx_ref[rs, 0:1])
            s1l.append(jnp.sum(wi * sxr, axis=1, keepdims=True))
            ti = jax.lax.dot_general(
                wi, c_ref[rs, :], (((1,), (0,)), ((), ())),
                preferred_element_type=jnp.float32)
            s2l.append(jnp.sum(ti * wi, axis=1, keepdims=True))
            cntl.append(jnp.broadcast_to(cnt_ref[i:i + 1, 0:1], (_DO, 1)))
        s1 = jnp.concatenate(s1l, axis=0)
        s2 = jnp.concatenate(s2l, axis=0)
        cnt = jnp.concatenate(cntl, axis=0)
        bb = bb_ref[:, 0:1]
        gb = gb_ref[:, 0:1]
        be = be_ref[:, 0:1]
        cmax = jnp.maximum(cnt, 1.0)
        s1y = s1 + cnt * bb
        s2y = s2 + 2.0 * bb * s1 + cnt * bb * bb
        mean = s1y / cmax
        var = s2y / cmax - mean * mean
        inv = jax.lax.rsqrt(var + _EPS)
        mx_ref[:, 0:1] = (m_ref[0][:, 0:1] + bb - mean) * (inv * gb) + be

    r = r_ref[0]
    acc = jnp.zeros((_DO, r.shape[1]), jnp.float32)
    for i in range(_NUM_RING):
        col = mx_ref[i * _DO:(i + 1) * _DO, 0:1]
        acc = jnp.where(r == i, col, acc)
    out_ref[0] = acc


def kernel(x, ring, W, b, gamma, beta):
    B_, D, N = x.shape
    ring3 = ring.reshape(B_, 1, N)
    wcat = W.reshape(_DR, D)
    bb = jnp.broadcast_to(b.reshape(_DR, 1), (_DR, 8))
    gb = jnp.broadcast_to(gamma.reshape(_DR, 1), (_DR, 8))
    be = jnp.broadcast_to(beta.reshape(_DR, 1), (_DR, 8))

    TN = 1024
    nt = N // TN
    cst = lambda shape: pl.BlockSpec(shape, lambda bi, ni: tuple(0 for _ in shape))
    M, SX, C, CNT = pl.pallas_call(
        _stats_kernel,
        grid=(B_, nt),
        in_specs=[
            pl.BlockSpec((1, D, TN), lambda bi, ni: (bi, 0, ni)),
            pl.BlockSpec((1, 1, TN), lambda bi, ni: (bi, 0, ni)),
            cst((_DR, D)),
        ],
        out_specs=[
            pl.BlockSpec((1, _DR, 8), lambda bi, ni: (bi, 0, 0)),
            cst((_NUM_RING * D, 8)),
            cst((_NUM_RING * D, D)),
            cst((8, 8)),
        ],
        out_shape=[
            jax.ShapeDtypeStruct((B_, _DR, 8), jnp.float32),
            jax.ShapeDtypeStruct((_NUM_RING * D, 8), jnp.float32),
            jax.ShapeDtypeStruct((_NUM_RING * D, D), jnp.float32),
            jax.ShapeDtypeStruct((8, 8), jnp.float32),
        ],
        compiler_params=pltpu.CompilerParams(
            dimension_semantics=("arbitrary", "arbitrary")),
    )(x, ring3, wcat)

    TN2 = 2048
    nt2 = N // TN2
    out = pl.pallas_call(
        _bcast_kernel,
        grid=(B_, nt2),
        in_specs=[
            pl.BlockSpec((1, _DR, 8), lambda bi, ni: (bi, 0, 0)),
            cst((_NUM_RING * D, 8)),
            cst((_NUM_RING * D, D)),
            cst((8, 8)),
            cst((_DR, D)),
            cst((_DR, 8)),
            cst((_DR, 8)),
            cst((_DR, 8)),
            pl.BlockSpec((1, 1, TN2), lambda bi, ni: (bi, 0, ni)),
        ],
        out_specs=pl.BlockSpec((1, _DO, TN2), lambda bi, ni: (bi, 0, ni)),
        out_shape=jax.ShapeDtypeStruct((B_, _DO, N), jnp.float32),
        scratch_shapes=[pltpu.VMEM((_DR, 8), jnp.float32)],
        compiler_params=pltpu.CompilerParams(
            dimension_semantics=("arbitrary", "arbitrary")),
    )(M, SX, C, CNT, wcat, bb, gb, be, ring3)
    return out


# speedup vs baseline: 1.6472x; 1.3237x over previous
import jax
import jax.numpy as jnp
from jax.experimental import pallas as pl
from jax.experimental.pallas import tpu as pltpu

_NUM_RING = 4
_EPS = 1e-5
_DO = 128
_DR = _NUM_RING * _DO
_NEG = -1e30


def _stats_kernel(x_ref, r_ref, w_ref, m_ref, sx_ref, c_ref, cnt_ref):
    b = pl.program_id(0)
    nt = pl.program_id(1)

    @pl.when(nt == 0)
    def _init_max():
        m_ref[0] = jnp.full(m_ref.shape[1:], _NEG, jnp.float32)

    @pl.when(jnp.logical_and(b == 0, nt == 0))
    def _init_sums():
        sx_ref[...] = jnp.zeros(sx_ref.shape, jnp.float32)
        c_ref[...] = jnp.zeros(c_ref.shape, jnp.float32)
        cnt_ref[...] = jnp.zeros(cnt_ref.shape, jnp.float32)

    xb = x_ref[0]
    z = jax.lax.dot_general(
        w_ref[...], xb, (((1,), (0,)), ((), ())),
        preferred_element_type=jnp.float32)
    r = r_ref[0]

    for i in range(_NUM_RING):
        sl = slice(i * _DO, (i + 1) * _DO)
        pmax = jnp.max(jnp.where(r == i, z[sl, :], _NEG), axis=1,
                       keepdims=True)
        m_ref[0, sl, :] = jnp.maximum(m_ref[0, sl, :], pmax)

    xbt = jnp.transpose(xb)
    for i in range(_NUM_RING):
        mask = r == i
        xm = jnp.where(mask, xb, 0.0)
        sxi = jnp.sum(xm, axis=1, keepdims=True)
        ci = jax.lax.dot_general(
            xm, xbt, (((1,), (0,)), ((), ())),
            preferred_element_type=jnp.float32)
        pc = jnp.sum(jnp.where(mask, 1.0, 0.0), axis=1, keepdims=True)
        rs = slice(64 * i, 64 * (i + 1))
        sx_ref[rs, 0:1] += sxi
        c_ref[rs, :] += ci
        cnt_ref[i:i + 1, 0:1] += pc


def _bcast_kernel(m_ref, sx_ref, c_ref, cnt_ref, w_ref, bb_ref, gb_ref,
                  be_ref, r_ref, out_ref, mx_ref):
    ni = pl.program_id(1)

    @pl.when(ni == 0)
    def _affine():
        w = w_ref[...]
        s1l, s2l, cntl = [], [], []
        for i in range(_NUM_RING):
            wi = w[i * _DO:(i + 1) * _DO, :]
            rs = slice(64 * i, 64 * (i + 1))
            sxr = jnp.transpose(sx_ref[rs, 0:1])
            s1l.append(jnp.sum(wi * sxr, axis=1, keepdims=True))
            ti = jax.lax.dot_general(
                wi, c_ref[rs, :], (((1,), (0,)), ((), ())),
                preferred_element_type=jnp.float32)
            s2l.append(jnp.sum(ti * wi, axis=1, keepdims=True))
            cntl.append(jnp.broadcast_to(cnt_ref[i:i + 1, 0:1], (_DO, 1)))
        s1 = jnp.concatenate(s1l, axis=0)
        s2 = jnp.concatenate(s2l, axis=0)
        cnt = jnp.concatenate(cntl, axis=0)
        bb = bb_ref[:, 0:1]
        gb = gb_ref[:, 0:1]
        be = be_ref[:, 0:1]
        cmax = jnp.maximum(cnt, 1.0)
        s1y = s1 + cnt * bb
        s2y = s2 + 2.0 * bb * s1 + cnt * bb * bb
        mean = s1y / cmax
        var = s2y / cmax - mean * mean
        inv = jax.lax.rsqrt(var + _EPS)
        mx_ref[:, 0:1] = (m_ref[0][:, 0:1] + bb - mean) * (inv * gb) + be

    r = r_ref[0]
    acc = jnp.zeros((_DO, r.shape[1]), jnp.float32)
    for i in range(_NUM_RING):
        col = mx_ref[i * _DO:(i + 1) * _DO, 0:1]
        acc = jnp.where(r == i, col, acc)
    out_ref[0] = acc


def kernel(x, ring, W, b, gamma, beta):
    B_, D, N = x.shape
    ring3 = ring.reshape(B_, 1, N)
    wcat = W.reshape(_DR, D)
    bb = jnp.broadcast_to(b.reshape(_DR, 1), (_DR, 8))
    gb = jnp.broadcast_to(gamma.reshape(_DR, 1), (_DR, 8))
    be = jnp.broadcast_to(beta.reshape(_DR, 1), (_DR, 8))

    TN = 2048
    nt = N // TN
    cst = lambda shape: pl.BlockSpec(shape, lambda bi, ni: tuple(0 for _ in shape))
    M, SX, C, CNT = pl.pallas_call(
        _stats_kernel,
        grid=(B_, nt),
        in_specs=[
            pl.BlockSpec((1, D, TN), lambda bi, ni: (bi, 0, ni)),
            pl.BlockSpec((1, 1, TN), lambda bi, ni: (bi, 0, ni)),
            cst((_DR, D)),
        ],
        out_specs=[
            pl.BlockSpec((1, _DR, 8), lambda bi, ni: (bi, 0, 0)),
            cst((_NUM_RING * D, 8)),
            cst((_NUM_RING * D, D)),
            cst((8, 8)),
        ],
        out_shape=[
            jax.ShapeDtypeStruct((B_, _DR, 8), jnp.float32),
            jax.ShapeDtypeStruct((_NUM_RING * D, 8), jnp.float32),
            jax.ShapeDtypeStruct((_NUM_RING * D, D), jnp.float32),
            jax.ShapeDtypeStruct((8, 8), jnp.float32),
        ],
        compiler_params=pltpu.CompilerParams(
            dimension_semantics=("arbitrary", "arbitrary")),
    )(x, ring3, wcat)

    TN2 = 4096
    nt2 = N // TN2
    out = pl.pallas_call(
        _bcast_kernel,
        grid=(B_, nt2),
        in_specs=[
            pl.BlockSpec((1, _DR, 8), lambda bi, ni: (bi, 0, 0)),
            cst((_NUM_RING * D, 8)),
            cst((_NUM_RING * D, D)),
            cst((8, 8)),
            cst((_DR, D)),
            cst((_DR, 8)),
            cst((_DR, 8)),
            cst((_DR, 8)),
            pl.BlockSpec((1, 1, TN2), lambda bi, ni: (bi, 0, ni)),
        ],
        out_specs=pl.BlockSpec((1, _DO, TN2), lambda bi, ni: (bi, 0, ni)),
        out_shape=jax.ShapeDtypeStruct((B_, _DO, N), jnp.float32),
        scratch_shapes=[pltpu.VMEM((_DR, 8), jnp.float32)],
        compiler_params=pltpu.CompilerParams(
            dimension_semantics=("arbitrary", "arbitrary")),
    )(M, SX, C, CNT, wcat, bb, gb, be, ring3)
    return out


# speedup vs baseline: 1.8518x; 1.1242x over previous
import jax
import jax.numpy as jnp
from jax.experimental import pallas as pl
from jax.experimental.pallas import tpu as pltpu

_NUM_RING = 4
_EPS = 1e-5
_DO = 128
_DR = _NUM_RING * _DO
_NEG = -1e30


def _stats_kernel(x_ref, r_ref, w_ref, m_ref, sx_ref, c_ref, cnt_ref):
    b = pl.program_id(0)
    nt = pl.program_id(1)

    @pl.when(nt == 0)
    def _init_max():
        m_ref[0] = jnp.full(m_ref.shape[1:], _NEG, jnp.float32)

    @pl.when(jnp.logical_and(b == 0, nt == 0))
    def _init_sums():
        sx_ref[...] = jnp.zeros(sx_ref.shape, jnp.float32)
        c_ref[...] = jnp.zeros(c_ref.shape, jnp.float32)
        cnt_ref[...] = jnp.zeros(cnt_ref.shape, jnp.float32)

    xb = x_ref[0]
    z = jax.lax.dot_general(
        w_ref[...], xb, (((1,), (0,)), ((), ())),
        preferred_element_type=jnp.float32)
    r = r_ref[0]

    for i in range(_NUM_RING):
        sl = slice(i * _DO, (i + 1) * _DO)
        pmax = jnp.max(jnp.where(r == i, z[sl, :], _NEG), axis=1,
                       keepdims=True)
        m_ref[0, sl, :] = jnp.maximum(m_ref[0, sl, :], pmax)

    xbt = jnp.transpose(xb)
    for i in range(_NUM_RING):
        mask = r == i
        xm = jnp.where(mask, xb, 0.0)
        sxi = jnp.sum(xm, axis=1, keepdims=True)
        ci = jax.lax.dot_general(
            xm, xbt, (((1,), (0,)), ((), ())),
            preferred_element_type=jnp.float32)
        pc = jnp.sum(jnp.where(mask, 1.0, 0.0), axis=1, keepdims=True)
        rs = slice(64 * i, 64 * (i + 1))
        sx_ref[rs, 0:1] += sxi
        c_ref[rs, :] += ci
        cnt_ref[i:i + 1, 0:1] += pc


def _bcast_kernel(m_ref, sx_ref, c_ref, cnt_ref, w_ref, bb_ref, gb_ref,
                  be_ref, r_ref, out_ref, mx_ref):
    ni = pl.program_id(1)

    @pl.when(ni == 0)
    def _affine():
        w = w_ref[...]
        s1l, s2l, cntl = [], [], []
        for i in range(_NUM_RING):
            wi = w[i * _DO:(i + 1) * _DO, :]
            rs = slice(64 * i, 64 * (i + 1))
            sxr = jnp.transpose(sx_ref[rs, 0:1])
            s1l.append(jnp.sum(wi * sxr, axis=1, keepdims=True))
            ti = jax.lax.dot_general(
                wi, c_ref[rs, :], (((1,), (0,)), ((), ())),
                preferred_element_type=jnp.float32)
            s2l.append(jnp.sum(ti * wi, axis=1, keepdims=True))
            cntl.append(jnp.broadcast_to(cnt_ref[i:i + 1, 0:1], (_DO, 1)))
        s1 = jnp.concatenate(s1l, axis=0)
        s2 = jnp.concatenate(s2l, axis=0)
        cnt = jnp.concatenate(cntl, axis=0)
        bb = bb_ref[:, 0:1]
        gb = gb_ref[:, 0:1]
        be = be_ref[:, 0:1]
        cmax = jnp.maximum(cnt, 1.0)
        s1y = s1 + cnt * bb
        s2y = s2 + 2.0 * bb * s1 + cnt * bb * bb
        mean = s1y / cmax
        var = s2y / cmax - mean * mean
        inv = jax.lax.rsqrt(var + _EPS)
        mx_ref[:, 0:1] = (m_ref[0][:, 0:1] + bb - mean) * (inv * gb) + be

    r = r_ref[0]
    acc = jnp.zeros((_DO, r.shape[1]), jnp.float32)
    for i in range(_NUM_RING):
        col = mx_ref[i * _DO:(i + 1) * _DO, 0:1]
        acc = jnp.where(r == i, col, acc)
    out_ref[0] = acc


def kernel(x, ring, W, b, gamma, beta):
    B_, D, N = x.shape
    ring3 = ring.reshape(B_, 1, N)
    wcat = W.reshape(_DR, D)
    bb = jnp.broadcast_to(b.reshape(_DR, 1), (_DR, 8))
    gb = jnp.broadcast_to(gamma.reshape(_DR, 1), (_DR, 8))
    be = jnp.broadcast_to(beta.reshape(_DR, 1), (_DR, 8))

    TN = 4096
    nt = N // TN
    cst = lambda shape: pl.BlockSpec(shape, lambda bi, ni: tuple(0 for _ in shape))
    M, SX, C, CNT = pl.pallas_call(
        _stats_kernel,
        grid=(B_, nt),
        in_specs=[
            pl.BlockSpec((1, D, TN), lambda bi, ni: (bi, 0, ni)),
            pl.BlockSpec((1, 1, TN), lambda bi, ni: (bi, 0, ni)),
            cst((_DR, D)),
        ],
        out_specs=[
            pl.BlockSpec((1, _DR, 8), lambda bi, ni: (bi, 0, 0)),
            cst((_NUM_RING * D, 8)),
            cst((_NUM_RING * D, D)),
            cst((8, 8)),
        ],
        out_shape=[
            jax.ShapeDtypeStruct((B_, _DR, 8), jnp.float32),
            jax.ShapeDtypeStruct((_NUM_RING * D, 8), jnp.float32),
            jax.ShapeDtypeStruct((_NUM_RING * D, D), jnp.float32),
            jax.ShapeDtypeStruct((8, 8), jnp.float32),
        ],
        compiler_params=pltpu.CompilerParams(
            dimension_semantics=("arbitrary", "arbitrary")),
    )(x, ring3, wcat)

    TN2 = 8192
    nt2 = N // TN2
    out = pl.pallas_call(
        _bcast_kernel,
        grid=(B_, nt2),
        in_specs=[
            pl.BlockSpec((1, _DR, 8), lambda bi, ni: (bi, 0, 0)),
            cst((_NUM_RING * D, 8)),
            cst((_NUM_RING * D, D)),
            cst((8, 8)),
            cst((_DR, D)),
            cst((_DR, 8)),
            cst((_DR, 8)),
            cst((_DR, 8)),
            pl.BlockSpec((1, 1, TN2), lambda bi, ni: (bi, 0, ni)),
        ],
        out_specs=pl.BlockSpec((1, _DO, TN2), lambda bi, ni: (bi, 0, ni)),
        out_shape=jax.ShapeDtypeStruct((B_, _DO, N), jnp.float32),
        scratch_shapes=[pltpu.VMEM((_DR, 8), jnp.float32)],
        compiler_params=pltpu.CompilerParams(
            dimension_semantics=("arbitrary", "arbitrary")),
    )(M, SX, C, CNT, wcat, bb, gb, be, ring3)
    return out


# speedup vs baseline: 1.9635x; 1.0604x over previous
import jax
import jax.numpy as jnp
from jax.experimental import pallas as pl
from jax.experimental.pallas import tpu as pltpu

_NUM_RING = 4
_EPS = 1e-5
_DO = 128
_DR = _NUM_RING * _DO
_NEG = -1e30


def _stats_kernel(x_ref, r_ref, w_ref, m_ref, sx_ref, c_ref, cnt_ref):
    b = pl.program_id(0)
    nt = pl.program_id(1)

    @pl.when(nt == 0)
    def _init_max():
        m_ref[0] = jnp.full(m_ref.shape[1:], _NEG, jnp.float32)

    @pl.when(jnp.logical_and(b == 0, nt == 0))
    def _init_sums():
        sx_ref[...] = jnp.zeros(sx_ref.shape, jnp.float32)
        c_ref[...] = jnp.zeros(c_ref.shape, jnp.float32)
        cnt_ref[...] = jnp.zeros(cnt_ref.shape, jnp.float32)

    xb = x_ref[0]
    z = jax.lax.dot_general(
        w_ref[...], xb, (((1,), (0,)), ((), ())),
        preferred_element_type=jnp.float32)
    r = r_ref[0]

    for i in range(_NUM_RING):
        sl = slice(i * _DO, (i + 1) * _DO)
        pmax = jnp.max(jnp.where(r == i, z[sl, :], _NEG), axis=1,
                       keepdims=True)
        m_ref[0, sl, :] = jnp.maximum(m_ref[0, sl, :], pmax)

    xbt = jnp.transpose(xb)
    for i in range(_NUM_RING):
        mask = r == i
        xm = jnp.where(mask, xb, 0.0)
        sxi = jnp.sum(xm, axis=1, keepdims=True)
        ci = jax.lax.dot_general(
            xm, xbt, (((1,), (0,)), ((), ())),
            preferred_element_type=jnp.float32)
        pc = jnp.sum(jnp.where(mask, 1.0, 0.0), axis=1, keepdims=True)
        rs = slice(64 * i, 64 * (i + 1))
        sx_ref[rs, 0:1] += sxi
        c_ref[rs, :] += ci
        cnt_ref[i:i + 1, 0:1] += pc


def _bcast_kernel(m_ref, sx_ref, c_ref, cnt_ref, w_ref, bb_ref, gb_ref,
                  be_ref, r_ref, out_ref, mx_ref):
    ni = pl.program_id(1)

    @pl.when(ni == 0)
    def _affine():
        w = w_ref[...]
        s1l, s2l, cntl = [], [], []
        for i in range(_NUM_RING):
            wi = w[i * _DO:(i + 1) * _DO, :]
            rs = slice(64 * i, 64 * (i + 1))
            sxr = jnp.transpose(sx_ref[rs, 0:1])
            s1l.append(jnp.sum(wi * sxr, axis=1, keepdims=True))
            ti = jax.lax.dot_general(
                wi, c_ref[rs, :], (((1,), (0,)), ((), ())),
                preferred_element_type=jnp.float32)
            s2l.append(jnp.sum(ti * wi, axis=1, keepdims=True))
            cntl.append(jnp.broadcast_to(cnt_ref[i:i + 1, 0:1], (_DO, 1)))
        s1 = jnp.concatenate(s1l, axis=0)
        s2 = jnp.concatenate(s2l, axis=0)
        cnt = jnp.concatenate(cntl, axis=0)
        bb = bb_ref[:, 0:1]
        gb = gb_ref[:, 0:1]
        be = be_ref[:, 0:1]
        cmax = jnp.maximum(cnt, 1.0)
        s1y = s1 + cnt * bb
        s2y = s2 + 2.0 * bb * s1 + cnt * bb * bb
        mean = s1y / cmax
        var = s2y / cmax - mean * mean
        inv = jax.lax.rsqrt(var + _EPS)
        mx_ref[:, 0:1] = (m_ref[0][:, 0:1] + bb - mean) * (inv * gb) + be

    r = r_ref[0]
    acc = jnp.zeros((_DO, r.shape[1]), jnp.float32)
    for i in range(_NUM_RING):
        col = mx_ref[i * _DO:(i + 1) * _DO, 0:1]
        acc = jnp.where(r == i, col, acc)
    out_ref[0] = acc


def kernel(x, ring, W, b, gamma, beta):
    B_, D, N = x.shape
    ring3 = ring.reshape(B_, 1, N)
    wcat = W.reshape(_DR, D)
    bb = jnp.broadcast_to(b.reshape(_DR, 1), (_DR, 8))
    gb = jnp.broadcast_to(gamma.reshape(_DR, 1), (_DR, 8))
    be = jnp.broadcast_to(beta.reshape(_DR, 1), (_DR, 8))

    TN = 8192
    nt = N // TN
    cst = lambda shape: pl.BlockSpec(shape, lambda bi, ni: tuple(0 for _ in shape))
    M, SX, C, CNT = pl.pallas_call(
        _stats_kernel,
        grid=(B_, nt),
        in_specs=[
            pl.BlockSpec((1, D, TN), lambda bi, ni: (bi, 0, ni)),
            pl.BlockSpec((1, 1, TN), lambda bi, ni: (bi, 0, ni)),
            cst((_DR, D)),
        ],
        out_specs=[
            pl.BlockSpec((1, _DR, 8), lambda bi, ni: (bi, 0, 0)),
            cst((_NUM_RING * D, 8)),
            cst((_NUM_RING * D, D)),
            cst((8, 8)),
        ],
        out_shape=[
            jax.ShapeDtypeStruct((B_, _DR, 8), jnp.float32),
            jax.ShapeDtypeStruct((_NUM_RING * D, 8), jnp.float32),
            jax.ShapeDtypeStruct((_NUM_RING * D, D), jnp.float32),
            jax.ShapeDtypeStruct((8, 8), jnp.float32),
        ],
        compiler_params=pltpu.CompilerParams(
            dimension_semantics=("arbitrary", "arbitrary")),
    )(x, ring3, wcat)

    TN2 = 8192
    nt2 = N // TN2
    out = pl.pallas_call(
        _bcast_kernel,
        grid=(B_, nt2),
        in_specs=[
            pl.BlockSpec((1, _DR, 8), lambda bi, ni: (bi, 0, 0)),
            cst((_NUM_RING * D, 8)),
            cst((_NUM_RING * D, D)),
            cst((8, 8)),
            cst((_DR, D)),
            cst((_DR, 8)),
            cst((_DR, 8)),
            cst((_DR, 8)),
            pl.BlockSpec((1, 1, TN2), lambda bi, ni: (bi, 0, ni)),
        ],
        out_specs=pl.BlockSpec((1, _DO, TN2), lambda bi, ni: (bi, 0, ni)),
        out_shape=jax.ShapeDtypeStruct((B_, _DO, N), jnp.float32),
        scratch_shapes=[pltpu.VMEM((_DR, 8), jnp.float32)],
        compiler_params=pltpu.CompilerParams(
            dimension_semantics=("arbitrary", "arbitrary")),
    )(M, SX, C, CNT, wcat, bb, gb, be, ring3)
    return out
